# baseline (speedup 1.0000x reference)
import functools

import jax
import jax.numpy as jnp
from jax.experimental import pallas as pl
from jax.experimental.pallas import tpu as pltpu
from jax.experimental.pallas import tpu_sc as plsc

_N = 8192
_P = 256
_RADIUS2 = 2500.0
_INV_2SIG2 = 1.0 / (2.0 * 25.0 * 25.0)
_YGAP = 51.5

_B = 1024
_NB = _N // _B
_NB2 = _NB * _NB


def _gather_rows(table, idx):
    info = plsc.get_sparse_core_info()
    nw = info.num_cores * info.num_subcores
    b_per_w = _N // nw
    mesh = plsc.VectorSubcoreMesh(core_axis_name="c", subcore_axis_name="s")

    @functools.partial(
        pl.kernel, mesh=mesh,
        out_type=jax.ShapeDtypeStruct((_N, _P), jnp.float32),
        scratch_types=[
            pltpu.VMEM((b_per_w,), jnp.int32),
            pltpu.VMEM((b_per_w, _P), jnp.float32),
            pltpu.SemaphoreType.DMA,
        ],
    )
    def k(table_hbm, idx_hbm, out_hbm, idx_v, rows_v, sem):
        wid = jax.lax.axis_index("s") * info.num_cores + jax.lax.axis_index("c")
        base = wid * b_per_w
        pltpu.sync_copy(idx_hbm.at[pl.ds(base, b_per_w)], idx_v)
        pltpu.async_copy(table_hbm.at[idx_v], rows_v, sem).wait()
        pltpu.sync_copy(rows_v, out_hbm.at[pl.ds(base, b_per_w)])

    return k(table, idx)


def _loss_kernel(cnt_ref, bi_ref, bj_ref,
                 ci_ref, pi_ref, cj_ref, pj_ref, out_ref,
                 q_acc, w2_acc):
    g = pl.program_id(0)
    cnt = cnt_ref[0]
    i = bi_ref[g]
    start = jnp.logical_or(g == 0, bi_ref[jnp.maximum(g - 1, 0)] != i)
    end = jnp.logical_or(g == cnt - 1,
                         bi_ref[jnp.minimum(g + 1, _NB2 - 1)] != i)

    @pl.when(g == 0)
    def _zero_out():
        out_ref[...] = jnp.zeros((1, 1), jnp.float32)

    @pl.when(start)
    def _init():
        q_acc[...] = jnp.zeros_like(q_acc)
        w2_acc[...] = jnp.zeros_like(w2_acc)

    ci = ci_ref[...]
    cj = cj_ref[...]
    c2i = jnp.sum(ci * ci, axis=1, keepdims=True)
    c2j = jnp.sum(cj * cj, axis=1)
    dot = jax.lax.dot_general(
        ci, cj, (((1,), (1,)), ((), ())),
        preferred_element_type=jnp.float32)
    d2 = c2i + c2j[None, :] - 2.0 * dot
    mask = (d2 > 0.0) & (d2 <= _RADIUS2)
    w = jnp.where(mask, jnp.exp(-d2 * _INV_2SIG2), 0.0)

    pj = pj_ref[...]
    p2j = jnp.sum(pj * pj, axis=1, keepdims=True)
    q_acc[...] += jax.lax.dot_general(
        w, pj, (((1,), (0,)), ((), ())),
        preferred_element_type=jnp.float32)
    m2 = jnp.concatenate([p2j, jnp.ones_like(p2j)], axis=1)
    w2_acc[...] += jax.lax.dot_general(
        w, m2, (((1,), (0,)), ((), ())),
        preferred_element_type=jnp.float32)

    @pl.when(end)
    def _finalize():
        pi = pi_ref[...]
        p2i = jnp.sum(pi * pi, axis=1, keepdims=True)
        pq = jnp.sum(pi * q_acc[...], axis=1, keepdims=True)
        wsum = w2_acc[:, 1:2]
        num = p2i * wsum + w2_acc[:, 0:1] - 2.0 * pq
        denom = jnp.where(wsum > 0.0, wsum, 1.0)
        contrib = jnp.sum(num / denom)
        out_ref[...] += jnp.full((1, 1), contrib, jnp.float32)


def kernel(coords, identity_probs):
    q = jnp.clip((coords[:, 1] * (262144.0 / 1000.0)).astype(jnp.int32),
                 0, 262143)
    key = jnp.sort((q << 13) | jnp.arange(_N, dtype=jnp.int32))
    perm = key & 8191
    cs = jnp.take(coords, perm, axis=0)
    ps = _gather_rows(identity_probs, perm)

    ys = cs[:, 1].reshape(_NB, _B)
    ylo = jnp.min(ys, axis=1)
    yhi = jnp.max(ys, axis=1)
    gap = jnp.maximum(ylo[None, :] - yhi[:, None],
                      ylo[:, None] - yhi[None, :])
    active = (gap <= _YGAP).reshape(-1)
    count = jnp.sum(active.astype(jnp.int32))
    cnt = count.reshape(1)
    order = jnp.argsort(~active, stable=True).astype(jnp.int32)
    order = jnp.where(jnp.arange(_NB2) < count, order, order[count - 1])
    bi = order // _NB
    bj = order % _NB

    grid_spec = pltpu.PrefetchScalarGridSpec(
        num_scalar_prefetch=3,
        grid=(count,),
        in_specs=[
            pl.BlockSpec((_B, 2), lambda g, c, bi, bj: (bi[g], 0)),
            pl.BlockSpec((_B, _P), lambda g, c, bi, bj: (bi[g], 0)),
            pl.BlockSpec((_B, 2), lambda g, c, bi, bj: (bj[g], 0)),
            pl.BlockSpec((_B, _P), lambda g, c, bi, bj: (bj[g], 0)),
        ],
        out_specs=pl.BlockSpec((1, 1), lambda g, c, bi, bj: (0, 0)),
        scratch_shapes=[
            pltpu.VMEM((_B, _P), jnp.float32),
            pltpu.VMEM((_B, 2), jnp.float32),
        ],
    )
    out = pl.pallas_call(
        _loss_kernel,
        grid_spec=grid_spec,
        out_shape=jax.ShapeDtypeStruct((1, 1), jnp.float32),
    )(cnt, bi, bj, cs, ps, cs, ps)
    return out[0, 0] / _N


# speedup vs baseline: 1.0440x; 1.0440x over previous
import functools

import jax
import jax.numpy as jnp
from jax.experimental import pallas as pl
from jax.experimental.pallas import tpu as pltpu
from jax.experimental.pallas import tpu_sc as plsc

_N = 8192
_P = 256
_RADIUS2 = 2500.0
_INV_2SIG2 = 1.0 / (2.0 * 25.0 * 25.0)
_YGAP = 51.5

_B = 512
_NB = _N // _B
_NB2 = _NB * _NB


def _gather_rows(table, idx):
    info = plsc.get_sparse_core_info()
    nw = info.num_cores * info.num_subcores
    b_per_w = _N // nw
    mesh = plsc.VectorSubcoreMesh(core_axis_name="c", subcore_axis_name="s")

    @functools.partial(
        pl.kernel, mesh=mesh,
        out_type=jax.ShapeDtypeStruct((_N, _P), jnp.float32),
        scratch_types=[
            pltpu.VMEM((b_per_w,), jnp.int32),
            pltpu.VMEM((b_per_w, _P), jnp.float32),
            pltpu.SemaphoreType.DMA,
        ],
    )
    def k(table_hbm, idx_hbm, out_hbm, idx_v, rows_v, sem):
        wid = jax.lax.axis_index("s") * info.num_cores + jax.lax.axis_index("c")
        base = wid * b_per_w
        pltpu.sync_copy(idx_hbm.at[pl.ds(base, b_per_w)], idx_v)
        pltpu.async_copy(table_hbm.at[idx_v], rows_v, sem).wait()
        pltpu.sync_copy(rows_v, out_hbm.at[pl.ds(base, b_per_w)])

    return k(table, idx)


def _loss_kernel(cnt_ref, bi_ref, bj_ref,
                 ci_ref, pi_ref, cj_ref, pj_ref, out_ref,
                 q_acc, w2_acc):
    g = pl.program_id(0)
    cu = cnt_ref[0]
    i = bi_ref[g]
    j = bj_ref[g]
    ri = pl.ds(i * _B, _B)
    rj = pl.ds(j * _B, _B)

    @pl.when(g == 0)
    def _zero():
        out_ref[...] = jnp.zeros((1, 1), jnp.float32)
        q_acc[...] = jnp.zeros_like(q_acc)
        w2_acc[...] = jnp.zeros_like(w2_acc)

    @pl.when(g < cu)
    def _pair_step():
        ci = ci_ref[...]
        cj = cj_ref[...]
        c2i = jnp.sum(ci * ci, axis=1, keepdims=True)
        c2j = jnp.sum(cj * cj, axis=1)
        dot = jax.lax.dot_general(
            ci, cj, (((1,), (1,)), ((), ())),
            preferred_element_type=jnp.float32)
        d2 = c2i + c2j[None, :] - 2.0 * dot
        mask = (d2 > 0.0) & (d2 <= _RADIUS2)
        w = jnp.where(mask, jnp.exp(-d2 * _INV_2SIG2), 0.0)

        pj = pj_ref[...]
        p2j = jnp.sum(pj * pj, axis=1, keepdims=True)
        q_acc[ri, :] += jax.lax.dot_general(
            w, pj, (((1,), (0,)), ((), ())),
            preferred_element_type=jnp.float32)
        m2 = jnp.concatenate([p2j, jnp.ones_like(p2j)], axis=1)
        w2_acc[ri, :] += jax.lax.dot_general(
            w, m2, (((1,), (0,)), ((), ())),
            preferred_element_type=jnp.float32)

        @pl.when(i != j)
        def _mirror():
            pi = pi_ref[...]
            p2i = jnp.sum(pi * pi, axis=1, keepdims=True)
            q_acc[rj, :] += jax.lax.dot_general(
                w, pi, (((0,), (0,)), ((), ())),
                preferred_element_type=jnp.float32)
            m2i = jnp.concatenate([p2i, jnp.ones_like(p2i)], axis=1)
            w2_acc[rj, :] += jax.lax.dot_general(
                w, m2i, (((0,), (0,)), ((), ())),
                preferred_element_type=jnp.float32)

    @pl.when(g >= cu)
    def _finalize():
        pi = pi_ref[...]
        p2i = jnp.sum(pi * pi, axis=1, keepdims=True)
        pq = jnp.sum(pi * q_acc[ri, :], axis=1, keepdims=True)
        wsum = w2_acc[ri, 1:2]
        num = p2i * wsum + w2_acc[ri, 0:1] - 2.0 * pq
        denom = jnp.where(wsum > 0.0, wsum, 1.0)
        contrib = jnp.sum(num / denom)
        out_ref[...] += jnp.full((1, 1), contrib, jnp.float32)


def kernel(coords, identity_probs):
    q = jnp.clip((coords[:, 1] * (262144.0 / 1000.0)).astype(jnp.int32),
                 0, 262143)
    key = jnp.sort((q << 13) | jnp.arange(_N, dtype=jnp.int32))
    perm = key & 8191
    cs = jnp.take(coords, perm, axis=0)
    ps = _gather_rows(identity_probs, perm)

    ys = cs[:, 1].reshape(_NB, _B)
    ylo = jnp.min(ys, axis=1)
    yhi = jnp.max(ys, axis=1)
    gap = jnp.maximum(ylo[None, :] - yhi[:, None],
                      ylo[:, None] - yhi[None, :])
    iu = jnp.arange(_NB)
    upper = iu[:, None] <= iu[None, :]
    active = ((gap <= _YGAP) & upper).reshape(-1)
    count = jnp.sum(active.astype(jnp.int32))
    cnt = count.reshape(1)
    order = jnp.argsort(~active, stable=True).astype(jnp.int32)
    r = jnp.arange(_NB2, dtype=jnp.int32)
    fin = jnp.clip(r - count, 0, _NB - 1)
    is_pair = r < count
    bi = jnp.where(is_pair, order // _NB, fin)
    bj = jnp.where(is_pair, order % _NB, fin)

    grid_spec = pltpu.PrefetchScalarGridSpec(
        num_scalar_prefetch=3,
        grid=(count + _NB,),
        in_specs=[
            pl.BlockSpec((_B, 2), lambda g, c, bi, bj: (bi[g], 0)),
            pl.BlockSpec((_B, _P), lambda g, c, bi, bj: (bi[g], 0)),
            pl.BlockSpec((_B, 2), lambda g, c, bi, bj: (bj[g], 0)),
            pl.BlockSpec((_B, _P), lambda g, c, bi, bj: (bj[g], 0)),
        ],
        out_specs=pl.BlockSpec((1, 1), lambda g, c, bi, bj: (0, 0)),
        scratch_shapes=[
            pltpu.VMEM((_N, _P), jnp.float32),
            pltpu.VMEM((_N, 2), jnp.float32),
        ],
    )
    out = pl.pallas_call(
        _loss_kernel,
        grid_spec=grid_spec,
        out_shape=jax.ShapeDtypeStruct((1, 1), jnp.float32),
    )(cnt, bi, bj, cs, ps, cs, ps)
    return out[0, 0] / _N


# speedup vs baseline: 1.1250x; 1.0776x over previous
import functools

import jax
import jax.numpy as jnp
from jax.experimental import pallas as pl
from jax.experimental.pallas import tpu as pltpu
from jax.experimental.pallas import tpu_sc as plsc

_N = 8192
_P = 256
_RADIUS2 = 2500.0
_INV_2SIG2 = 1.0 / (2.0 * 25.0 * 25.0)
_YGAP = 51.5

_B = 512
_NB = _N // _B
_NB2 = _NB * _NB


def _gather_rows(table, idx):
    info = plsc.get_sparse_core_info()
    nw = info.num_cores * info.num_subcores
    b_per_w = _N // nw
    mesh = plsc.VectorSubcoreMesh(core_axis_name="c", subcore_axis_name="s")

    @functools.partial(
        pl.kernel, mesh=mesh,
        out_type=jax.ShapeDtypeStruct((_N, _P), jnp.float32),
        scratch_types=[
            pltpu.VMEM((b_per_w,), jnp.int32),
            pltpu.VMEM((b_per_w, _P), jnp.float32),
            pltpu.SemaphoreType.DMA,
        ],
    )
    def k(table_hbm, idx_hbm, out_hbm, idx_v, rows_v, sem):
        wid = jax.lax.axis_index("s") * info.num_cores + jax.lax.axis_index("c")
        base = wid * b_per_w
        pltpu.sync_copy(idx_hbm.at[pl.ds(base, b_per_w)], idx_v)
        pltpu.async_copy(table_hbm.at[idx_v], rows_v, sem).wait()
        pltpu.sync_copy(rows_v, out_hbm.at[pl.ds(base, b_per_w)])

    return k(table, idx)


def _loss_kernel(cnt_ref, bi_ref, bj_ref,
                 ci_ref, pi_ref, cj_ref, pj_ref, out_ref,
                 q_acc, w2_acc):
    g = pl.program_id(0)
    cnt = cnt_ref[0]
    i = bi_ref[g]
    start = jnp.logical_or(g == 0, bi_ref[jnp.maximum(g - 1, 0)] != i)
    end = jnp.logical_or(g == cnt - 1,
                         bi_ref[jnp.minimum(g + 1, _NB2 - 1)] != i)

    @pl.when(g == 0)
    def _zero_out():
        out_ref[...] = jnp.zeros((1, 1), jnp.float32)

    @pl.when(start)
    def _init():
        q_acc[...] = jnp.zeros_like(q_acc)
        w2_acc[...] = jnp.zeros_like(w2_acc)

    ci = ci_ref[...]
    cj = cj_ref[...]
    c2i = jnp.sum(ci * ci, axis=1, keepdims=True)
    c2j = jnp.sum(cj * cj, axis=1)
    dot = jax.lax.dot_general(
        ci, cj, (((1,), (1,)), ((), ())),
        preferred_element_type=jnp.float32)
    d2 = c2i + c2j[None, :] - 2.0 * dot
    mask = (d2 > 0.0) & (d2 <= _RADIUS2)
    w = jnp.where(mask, jnp.exp(-d2 * _INV_2SIG2), 0.0)

    pj = pj_ref[...]
    p2j = jnp.sum(pj * pj, axis=1, keepdims=True)
    q_acc[...] += jax.lax.dot_general(
        w, pj, (((1,), (0,)), ((), ())),
        preferred_element_type=jnp.float32)
    m2 = jnp.concatenate([p2j, jnp.ones_like(p2j)], axis=1)
    w2_acc[...] += jax.lax.dot_general(
        w, m2, (((1,), (0,)), ((), ())),
        preferred_element_type=jnp.float32)

    @pl.when(end)
    def _finalize():
        pi = pi_ref[...]
        p2i = jnp.sum(pi * pi, axis=1, keepdims=True)
        pq = jnp.sum(pi * q_acc[...], axis=1, keepdims=True)
        wsum = w2_acc[:, 1:2]
        num = p2i * wsum + w2_acc[:, 0:1] - 2.0 * pq
        denom = jnp.where(wsum > 0.0, wsum, 1.0)
        contrib = jnp.sum(num / denom)
        out_ref[...] += jnp.full((1, 1), contrib, jnp.float32)


def kernel(coords, identity_probs):
    q = jnp.clip((coords[:, 1] * (262144.0 / 1000.0)).astype(jnp.int32),
                 0, 262143)
    key = jnp.sort((q << 13) | jnp.arange(_N, dtype=jnp.int32))
    perm = key & 8191
    cs = jnp.take(coords, perm, axis=0)
    ps = _gather_rows(identity_probs, perm)

    qs = (key >> 13).reshape(_NB, _B).astype(jnp.float32) * (1000.0 / 262144.0)
    ylo = qs[:, 0]
    yhi = qs[:, -1] + (1000.0 / 262144.0)
    gap = jnp.maximum(ylo[None, :] - yhi[:, None],
                      ylo[:, None] - yhi[None, :])
    active = (gap <= _YGAP).reshape(-1)
    acti = active.astype(jnp.int32)
    pos = jnp.cumsum(acti)
    count = pos[-1]
    cnt = count.reshape(1)
    r = jnp.arange(_NB2, dtype=jnp.int32)
    scat = jnp.where(active, pos - 1, _NB2)
    order = jnp.zeros((_NB2,), jnp.int32).at[scat].set(r, mode="drop")
    order = jnp.where(r < count, order, order[count - 1])
    bi = order // _NB
    bj = order % _NB

    grid_spec = pltpu.PrefetchScalarGridSpec(
        num_scalar_prefetch=3,
        grid=(count,),
        in_specs=[
            pl.BlockSpec((_B, 2), lambda g, c, bi, bj: (bi[g], 0)),
            pl.BlockSpec((_B, _P), lambda g, c, bi, bj: (bi[g], 0)),
            pl.BlockSpec((_B, 2), lambda g, c, bi, bj: (bj[g], 0)),
            pl.BlockSpec((_B, _P), lambda g, c, bi, bj: (bj[g], 0)),
        ],
        out_specs=pl.BlockSpec((1, 1), lambda g, c, bi, bj: (0, 0)),
        scratch_shapes=[
            pltpu.VMEM((_B, _P), jnp.float32),
            pltpu.VMEM((_B, 2), jnp.float32),
        ],
    )
    out = pl.pallas_call(
        _loss_kernel,
        grid_spec=grid_spec,
        out_shape=jax.ShapeDtypeStruct((1, 1), jnp.float32),
    )(cnt, bi, bj, cs, ps, cs, ps)
    return out[0, 0] / _N
